# baseline (speedup 1.0000x reference)
import jax
import jax.numpy as jnp
import numpy as np
from jax.experimental import pallas as pl
from jax.experimental.pallas import tpu as pltpu

DROP_P = 0.1
KEY_LO = 42
KEY_HI = 0
T = 2048

ROWS = 8192
BR = 512
NBUF = 6


def _np_rotl(x, r):
    return ((x << np.uint32(r)) | (x >> np.uint32(32 - r))).astype(np.uint32)


def _np_dropped_mask():
    i = np.arange(ROWS, dtype=np.uint32)
    k0, k1 = np.uint32(KEY_HI), np.uint32(KEY_LO)
    ks = [k0, k1, np.uint32(k0 ^ k1 ^ np.uint32(0x1BD11BDA))]
    x0 = np.zeros_like(i) + ks[0]
    x1 = (i + ks[1]).astype(np.uint32)
    rotations = [(13, 15, 26, 6), (17, 29, 16, 24)]
    for rnd in range(5):
        for r in rotations[rnd % 2]:
            x0 = (x0 + x1).astype(np.uint32)
            x1 = _np_rotl(x1, r)
            x1 = (x1 ^ x0).astype(np.uint32)
        x0 = (x0 + ks[(rnd + 1) % 3]).astype(np.uint32)
        x1 = (x1 + ks[(rnd + 2) % 3] + np.uint32(rnd + 1)).astype(np.uint32)
    bits = x0 ^ x1
    u = ((bits >> np.uint32(9)) | np.uint32(0x3F800000)).view(np.float32)
    u = u - np.float32(1.0)
    drop = u < np.float32(DROP_P)
    drop[i % np.uint32(T) == 0] = False
    return drop


_DROPPED_BY_BLOCK = [
    np.nonzero(_np_dropped_mask()[k * BR:(k + 1) * BR])[0].tolist()
    for k in range(ROWS // BR)
]


def _body(x_hbm, o_hbm, buf, *sems):
    F = x_hbm.shape[1]
    sems_in = sems[:NBUF]
    sems_out = sems[NBUF:]
    nblocks = ROWS // BR
    zrow = jnp.zeros((1, F), jnp.float32)

    def in_copy(k):
        b = k % NBUF
        return pltpu.make_async_copy(
            x_hbm.at[pl.ds(k * BR, BR)], buf.at[b], sems_in[b]
        )

    def out_copy(k):
        b = k % NBUF
        return pltpu.make_async_copy(
            buf.at[b], o_hbm.at[pl.ds(k * BR, BR)], sems_out[b]
        )

    for j in range(NBUF):
        in_copy(j).start()
    for k in range(nblocks):
        b = k % NBUF
        in_copy(k).wait()
        for r in _DROPPED_BY_BLOCK[k]:
            buf[b, r:r + 1, :] = zrow
        out_copy(k).start()
        if k + NBUF < nblocks:
            out_copy(k).wait()
            in_copy(k + NBUF).start()
    for k in range(nblocks - NBUF, nblocks):
        out_copy(k).wait()


def kernel(x):
    B, t, F = x.shape
    x2 = x.reshape(ROWS, F)
    out = pl.pallas_call(
        _body,
        in_specs=[pl.BlockSpec(memory_space=pltpu.MemorySpace.HBM)],
        out_specs=pl.BlockSpec(memory_space=pltpu.MemorySpace.HBM),
        out_shape=jax.ShapeDtypeStruct((ROWS, F), jnp.float32),
        scratch_shapes=[pltpu.VMEM((NBUF, BR, F), jnp.float32)]
        + [pltpu.SemaphoreType.DMA] * (2 * NBUF),
    )(x2)
    return out.reshape(B, t, F)


# speedup vs baseline: 1.0002x; 1.0002x over previous
import jax
import jax.numpy as jnp
import numpy as np
from jax.experimental import pallas as pl
from jax.experimental.pallas import tpu as pltpu

DROP_P = 0.1
KEY_LO = 42
KEY_HI = 0
T = 2048

ROWS = 8192
BR = 256
NBUF = 12


def _np_rotl(x, r):
    return ((x << np.uint32(r)) | (x >> np.uint32(32 - r))).astype(np.uint32)


def _np_dropped_mask():
    i = np.arange(ROWS, dtype=np.uint32)
    k0, k1 = np.uint32(KEY_HI), np.uint32(KEY_LO)
    ks = [k0, k1, np.uint32(k0 ^ k1 ^ np.uint32(0x1BD11BDA))]
    x0 = np.zeros_like(i) + ks[0]
    x1 = (i + ks[1]).astype(np.uint32)
    rotations = [(13, 15, 26, 6), (17, 29, 16, 24)]
    for rnd in range(5):
        for r in rotations[rnd % 2]:
            x0 = (x0 + x1).astype(np.uint32)
            x1 = _np_rotl(x1, r)
            x1 = (x1 ^ x0).astype(np.uint32)
        x0 = (x0 + ks[(rnd + 1) % 3]).astype(np.uint32)
        x1 = (x1 + ks[(rnd + 2) % 3] + np.uint32(rnd + 1)).astype(np.uint32)
    bits = x0 ^ x1
    u = ((bits >> np.uint32(9)) | np.uint32(0x3F800000)).view(np.float32)
    u = u - np.float32(1.0)
    drop = u < np.float32(DROP_P)
    drop[i % np.uint32(T) == 0] = False
    return drop


_DROPPED_BY_BLOCK = [
    np.nonzero(_np_dropped_mask()[k * BR:(k + 1) * BR])[0].tolist()
    for k in range(ROWS // BR)
]


def _body(x_hbm, o_hbm, buf, *sems):
    F = x_hbm.shape[1]
    sems_in = sems[:NBUF]
    sems_out = sems[NBUF:]
    nblocks = ROWS // BR
    zrow = jnp.zeros((1, F), jnp.float32)

    def in_copy(k):
        b = k % NBUF
        return pltpu.make_async_copy(
            x_hbm.at[pl.ds(k * BR, BR)], buf.at[b], sems_in[b]
        )

    def out_copy(k):
        b = k % NBUF
        return pltpu.make_async_copy(
            buf.at[b], o_hbm.at[pl.ds(k * BR, BR)], sems_out[b]
        )

    for j in range(NBUF):
        in_copy(j).start()
    for k in range(nblocks):
        b = k % NBUF
        in_copy(k).wait()
        for r in _DROPPED_BY_BLOCK[k]:
            buf[b, r:r + 1, :] = zrow
        out_copy(k).start()
        if k + NBUF < nblocks:
            out_copy(k).wait()
            in_copy(k + NBUF).start()
    for k in range(nblocks - NBUF, nblocks):
        out_copy(k).wait()


def kernel(x):
    B, t, F = x.shape
    x2 = x.reshape(ROWS, F)
    out = pl.pallas_call(
        _body,
        in_specs=[pl.BlockSpec(memory_space=pltpu.MemorySpace.HBM)],
        out_specs=pl.BlockSpec(memory_space=pltpu.MemorySpace.HBM),
        out_shape=jax.ShapeDtypeStruct((ROWS, F), jnp.float32),
        scratch_shapes=[pltpu.VMEM((NBUF, BR, F), jnp.float32)]
        + [pltpu.SemaphoreType.DMA] * (2 * NBUF),
    )(x2)
    return out.reshape(B, t, F)
